# baseline (speedup 1.0000x reference)
import functools

import jax
import jax.numpy as jnp
from jax.experimental import pallas as pl
from jax.experimental.pallas import tpu as pltpu

NUM_EXPERTS = 8
TOP_K = 2
HIDDEN = 1024
INTER = 1408
TOKENS = 2048
TM = 256


def _moe_body(sel_ref, rw_ref, x_ref, g_ref, u_ref, d_ref, out_ref):
    t = pl.program_id(0)
    e = pl.program_id(1)
    x = x_ref[...]
    g = g_ref[0]
    u = u_ref[0]
    dn = d_ref[0]

    gate_out = jax.lax.dot_general(
        x, g, (((1,), (1,)), ((), ())), preferred_element_type=jnp.float32)
    up_out = jax.lax.dot_general(
        x, u, (((1,), (1,)), ((), ())), preferred_element_type=jnp.float32)
    act = gate_out * jax.nn.sigmoid(gate_out) * up_out
    expert_out = jax.lax.dot_general(
        act, dn, (((1,), (1,)), ((), ())), preferred_element_type=jnp.float32)

    sel = sel_ref[pl.ds(t * TM, TM), :]
    rw = rw_ref[pl.ds(t * TM, TM), :]
    w = jnp.sum(rw * (sel == e).astype(jnp.float32), axis=1)
    contrib = expert_out * w[:, None]

    @pl.when(e == 0)
    def _init():
        out_ref[...] = contrib

    @pl.when(e > 0)
    def _acc():
        out_ref[...] += contrib


@jax.jit
def kernel(hidden_states, routing_weights, selected_experts, gate_proj, up_proj, down_proj):
    grid = (TOKENS // TM, NUM_EXPERTS)
    return pl.pallas_call(
        _moe_body,
        grid=grid,
        in_specs=[
            pl.BlockSpec((TOKENS, TOP_K), lambda t, e: (0, 0)),
            pl.BlockSpec((TOKENS, TOP_K), lambda t, e: (0, 0)),
            pl.BlockSpec((TM, HIDDEN), lambda t, e: (t, 0)),
            pl.BlockSpec((1, INTER, HIDDEN), lambda t, e: (e, 0, 0)),
            pl.BlockSpec((1, INTER, HIDDEN), lambda t, e: (e, 0, 0)),
            pl.BlockSpec((1, HIDDEN, INTER), lambda t, e: (e, 0, 0)),
        ],
        out_specs=pl.BlockSpec((TM, HIDDEN), lambda t, e: (t, 0)),
        out_shape=jax.ShapeDtypeStruct((TOKENS, HIDDEN), jnp.float32),
        compiler_params=pltpu.CompilerParams(
            dimension_semantics=("parallel", "arbitrary")),
    )(selected_experts.astype(jnp.int32), routing_weights, hidden_states,
      gate_proj, up_proj, down_proj)


# speedup vs baseline: 1.1026x; 1.1026x over previous
import functools

import jax
import jax.numpy as jnp
from jax.experimental import pallas as pl
from jax.experimental.pallas import tpu as pltpu

NUM_EXPERTS = 8
TOP_K = 2
HIDDEN = 1024
INTER = 1408
TOKENS = 2048
FLAT = TOKENS * TOP_K

TM = 128
NPAD = FLAT + NUM_EXPERTS * TM
GRID = NPAD // TM


def _gemm_body(e_of_ref, x_ref, g_ref, u_ref, d_ref, w_ref, y_ref):
    x = x_ref[...].astype(jnp.bfloat16)
    g = g_ref[0].astype(jnp.bfloat16)
    u = u_ref[0].astype(jnp.bfloat16)
    dn = d_ref[0].astype(jnp.bfloat16)

    gate_out = jax.lax.dot_general(
        x, g, (((1,), (1,)), ((), ())), preferred_element_type=jnp.float32)
    up_out = jax.lax.dot_general(
        x, u, (((1,), (1,)), ((), ())), preferred_element_type=jnp.float32)
    act = gate_out * jax.nn.sigmoid(gate_out) * up_out
    y = jax.lax.dot_general(
        act.astype(jnp.bfloat16), dn, (((1,), (1,)), ((), ())),
        preferred_element_type=jnp.float32)
    y_ref[...] = y * w_ref[0]


def _grouped_gemm(e_of_tile, x_sorted, gate_proj, up_proj, down_proj, w_sorted):
    w3 = w_sorted.reshape(GRID, TM, 1)
    grid_spec = pltpu.PrefetchScalarGridSpec(
        num_scalar_prefetch=1,
        grid=(GRID,),
        in_specs=[
            pl.BlockSpec((TM, HIDDEN), lambda t, e_of: (t, 0)),
            pl.BlockSpec((1, INTER, HIDDEN), lambda t, e_of: (e_of[t], 0, 0)),
            pl.BlockSpec((1, INTER, HIDDEN), lambda t, e_of: (e_of[t], 0, 0)),
            pl.BlockSpec((1, HIDDEN, INTER), lambda t, e_of: (e_of[t], 0, 0)),
            pl.BlockSpec((1, TM, 1), lambda t, e_of: (t, 0, 0)),
        ],
        out_specs=pl.BlockSpec((TM, HIDDEN), lambda t, e_of: (t, 0)),
    )
    return pl.pallas_call(
        _gemm_body,
        grid_spec=grid_spec,
        out_shape=jax.ShapeDtypeStruct((NPAD, HIDDEN), jnp.float32),
        compiler_params=pltpu.CompilerParams(
            dimension_semantics=("arbitrary",)),
    )(e_of_tile, x_sorted, gate_proj, up_proj, down_proj, w3)


@jax.jit
def kernel(hidden_states, routing_weights, selected_experts, gate_proj, up_proj, down_proj):
    flat_e = selected_experts.reshape(-1).astype(jnp.int32)
    flat_w = routing_weights.reshape(-1)
    flat_tok = jnp.arange(FLAT, dtype=jnp.int32) // TOP_K

    counts = jnp.sum(flat_e[:, None] == jnp.arange(NUM_EXPERTS)[None, :], axis=0)
    cpad = ((counts + TM - 1) // TM) * TM
    offs_p = jnp.concatenate([jnp.zeros((1,), jnp.int32),
                              jnp.cumsum(cpad).astype(jnp.int32)])
    offs_u = jnp.concatenate([jnp.zeros((1,), jnp.int32),
                              jnp.cumsum(counts).astype(jnp.int32)])

    order = jnp.argsort(flat_e, stable=True)
    e_sorted = flat_e[order]
    dest = offs_p[e_sorted] + (jnp.arange(FLAT, dtype=jnp.int32) - offs_u[e_sorted])

    row_sorted = jnp.zeros((NPAD,), jnp.int32).at[dest].set(flat_tok[order])
    w_sorted = jnp.zeros((NPAD,), jnp.float32).at[dest].set(flat_w[order])
    x_sorted = hidden_states[row_sorted]

    tile_starts = jnp.arange(GRID, dtype=jnp.int32) * TM
    e_of_tile = jnp.sum(tile_starts[:, None] >= offs_p[None, 1:NUM_EXPERTS],
                        axis=1).astype(jnp.int32)

    y = _grouped_gemm(e_of_tile, x_sorted, gate_proj, up_proj, down_proj, w_sorted)

    out = jnp.zeros((TOKENS, HIDDEN), jnp.float32).at[row_sorted].add(y)
    return out


# speedup vs baseline: 1.1113x; 1.0080x over previous
import functools

import jax
import jax.numpy as jnp
from jax import lax
from jax.experimental import pallas as pl
from jax.experimental.pallas import tpu as pltpu
from jax.experimental.pallas import tpu_sc as plsc

NUM_EXPERTS = 8
TOP_K = 2
HIDDEN = 1024
HHALF = HIDDEN // 2
INTER = 1408
TOKENS = 2048
FLAT = TOKENS * TOP_K

TM = 128
NPAD = FLAT + NUM_EXPERTS * TM
GRID = NPAD // TM
EOF_PAD = ((GRID + 15) // 16) * 16

L = 16
RPT = NPAD // 32
GCH = RPT // 2
CTOK = TOKENS // 32
CCH = 32


def _gather_body(row_hbm, hidden_hbm, x_hbm, idx_buf, gxbuf, sem):
    c = lax.axis_index("c")
    s = lax.axis_index("s")
    gbase = (c * 16 + s) * RPT
    for k in range(RPT // GCH):
        goff = pl.multiple_of(gbase + k * GCH, GCH)
        pltpu.sync_copy(row_hbm.at[pl.ds(goff, GCH)], idx_buf)
        pltpu.async_copy(hidden_hbm.at[idx_buf], gxbuf, sem).wait()
        pltpu.sync_copy(gxbuf, x_hbm.at[pl.ds(goff, GCH)])


_sc_gather = pl.kernel(
    _gather_body,
    out_type=[jax.ShapeDtypeStruct((NPAD, HIDDEN), jnp.float32)],
    mesh=plsc.VectorSubcoreMesh(core_axis_name="c", subcore_axis_name="s"),
    scratch_types=[
        pltpu.VMEM((GCH,), jnp.int32),
        pltpu.VMEM((GCH, HIDDEN), jnp.float32),
        pltpu.SemaphoreType.DMA,
    ],
)


def _combine_body(y_hbm, pos0_hbm, pos1_hbm, out_hbm,
                  idx0_buf, idx1_buf, b0, b1, sem):
    c = lax.axis_index("c")
    s = lax.axis_index("s")
    base = (c * 16 + s) * CTOK
    for k in range(CTOK // CCH):
        boff = pl.multiple_of(base + k * CCH, CCH)
        pltpu.sync_copy(pos0_hbm.at[pl.ds(boff, CCH)], idx0_buf)
        pltpu.sync_copy(pos1_hbm.at[pl.ds(boff, CCH)], idx1_buf)
        pltpu.async_copy(y_hbm.at[idx0_buf], b0, sem).wait()
        pltpu.async_copy(y_hbm.at[idx1_buf], b1, sem).wait()

        def arow(r, _):
            def acol(j, _):
                b0[r, pl.ds(j * L, L)] = (b0[r, pl.ds(j * L, L)]
                                          + b1[r, pl.ds(j * L, L)])
                return 0
            lax.fori_loop(0, HIDDEN // L, acol, 0)
            return 0

        lax.fori_loop(0, CCH, arow, 0)
        pltpu.sync_copy(b0, out_hbm.at[pl.ds(boff, CCH)])


_sc_combine = pl.kernel(
    _combine_body,
    out_type=[jax.ShapeDtypeStruct((TOKENS, HIDDEN), jnp.float32)],
    mesh=plsc.VectorSubcoreMesh(core_axis_name="c", subcore_axis_name="s"),
    scratch_types=[
        pltpu.VMEM((CCH,), jnp.int32),
        pltpu.VMEM((CCH,), jnp.int32),
        pltpu.VMEM((CCH, HIDDEN), jnp.float32),
        pltpu.VMEM((CCH, HIDDEN), jnp.float32),
        pltpu.SemaphoreType.DMA,
    ],
)


def _gemm_body(e_of_ref, x_ref, g_ref, u_ref, d_ref, w_ref, y_ref,
               gb_ref, ub_ref, db_ref):
    t = pl.program_id(0)
    new_expert = (t == 0) | (e_of_ref[t] != e_of_ref[jnp.maximum(t - 1, 0)])

    @pl.when(new_expert)
    def _cache():
        gb_ref[...] = g_ref[0].astype(jnp.bfloat16)
        ub_ref[...] = u_ref[0].astype(jnp.bfloat16)
        db_ref[...] = d_ref[0].astype(jnp.bfloat16)

    x = x_ref[...].astype(jnp.bfloat16)
    gate_out = jax.lax.dot_general(
        x, gb_ref[...], (((1,), (1,)), ((), ())),
        preferred_element_type=jnp.float32)
    up_out = jax.lax.dot_general(
        x, ub_ref[...], (((1,), (1,)), ((), ())),
        preferred_element_type=jnp.float32)
    act = gate_out * jax.nn.sigmoid(gate_out) * up_out
    y = jax.lax.dot_general(
        act.astype(jnp.bfloat16), db_ref[...], (((1,), (1,)), ((), ())),
        preferred_element_type=jnp.float32)
    y_ref[...] = y * w_ref[0]


def _grouped_gemm(e_of_tile, x_sorted, gate_proj, up_proj, down_proj, w_sorted):
    w3 = w_sorted.reshape(GRID, TM, 1)
    grid_spec = pltpu.PrefetchScalarGridSpec(
        num_scalar_prefetch=1,
        grid=(GRID,),
        in_specs=[
            pl.BlockSpec((TM, HIDDEN), lambda t, e_of: (t, 0)),
            pl.BlockSpec((1, INTER, HIDDEN), lambda t, e_of: (e_of[t], 0, 0)),
            pl.BlockSpec((1, INTER, HIDDEN), lambda t, e_of: (e_of[t], 0, 0)),
            pl.BlockSpec((1, HIDDEN, INTER), lambda t, e_of: (e_of[t], 0, 0)),
            pl.BlockSpec((1, TM, 1), lambda t, e_of: (t, 0, 0)),
        ],
        out_specs=pl.BlockSpec((TM, HIDDEN), lambda t, e_of: (t, 0)),
        scratch_shapes=[
            pltpu.VMEM((INTER, HIDDEN), jnp.bfloat16),
            pltpu.VMEM((INTER, HIDDEN), jnp.bfloat16),
            pltpu.VMEM((HIDDEN, INTER), jnp.bfloat16),
        ],
    )
    return pl.pallas_call(
        _gemm_body,
        grid_spec=grid_spec,
        out_shape=jax.ShapeDtypeStruct((NPAD, HIDDEN), jnp.float32),
        compiler_params=pltpu.CompilerParams(
            dimension_semantics=("arbitrary",)),
    )(e_of_tile, x_sorted, gate_proj, up_proj, down_proj, w3)


@jax.jit
def kernel(hidden_states, routing_weights, selected_experts, gate_proj, up_proj, down_proj):
    flat_e = selected_experts.reshape(-1).astype(jnp.int32)
    flat_w = routing_weights.reshape(-1)
    flat_tok = jnp.arange(FLAT, dtype=jnp.int32) // TOP_K

    onehot = (flat_e[:, None] == jnp.arange(NUM_EXPERTS)[None, :])
    counts = jnp.sum(onehot, axis=0, dtype=jnp.int32)
    cpad = ((counts + TM - 1) // TM) * TM
    offs_p = jnp.concatenate([jnp.zeros((1,), jnp.int32),
                              jnp.cumsum(cpad).astype(jnp.int32)])

    rank_all = jnp.cumsum(onehot.astype(jnp.int32), axis=0) - 1
    rank = jnp.sum(jnp.where(onehot, rank_all, 0), axis=1)
    dest = offs_p[flat_e] + rank

    row_sorted = jnp.zeros((NPAD,), jnp.int32).at[dest].set(flat_tok)
    w_sorted = jnp.zeros((NPAD,), jnp.float32).at[dest].set(flat_w)

    tile_starts = jnp.arange(GRID, dtype=jnp.int32) * TM
    e_of_tile = jnp.sum(tile_starts[:, None] >= offs_p[None, 1:NUM_EXPERTS],
                        axis=1).astype(jnp.int32)

    pos = dest.reshape(TOKENS, TOP_K)

    (x_sorted,) = _sc_gather(row_sorted, hidden_states)

    y = _grouped_gemm(e_of_tile, x_sorted, gate_proj, up_proj,
                      down_proj, w_sorted)

    (out,) = _sc_combine(y, pos[:, 0], pos[:, 1])
    return out


# speedup vs baseline: 1.1945x; 1.0748x over previous
import functools

import jax
import jax.numpy as jnp
from jax import lax
from jax.experimental import pallas as pl
from jax.experimental.pallas import tpu as pltpu
from jax.experimental.pallas import tpu_sc as plsc

NUM_EXPERTS = 8
TOP_K = 2
HIDDEN = 1024
HHALF = HIDDEN // 2
INTER = 1408
TOKENS = 2048
FLAT = TOKENS * TOP_K

TM = 128
NPAD = FLAT + NUM_EXPERTS * TM
GRID = NPAD // TM
EOF_PAD = ((GRID + 15) // 16) * 16

L = 16
RPT = NPAD // 32
GCH = RPT // 4
CTOK = TOKENS // 32
CCH = 16


def _gather_body(row_hbm, hidden_hbm, x_hbm, idx_buf, bufa, bufb,
                 sga, sgb, swa, swb):
    c = lax.axis_index("c")
    s = lax.axis_index("s")
    gbase = (c * 16 + s) * RPT
    offs = []
    for k in range(RPT // GCH):
        off_k = pl.multiple_of(gbase + k * GCH, GCH)
        offs.append(off_k)
        pltpu.sync_copy(row_hbm.at[pl.ds(off_k, GCH)], idx_buf.at[k])
    g0 = pltpu.async_copy(hidden_hbm.at[idx_buf.at[0]], bufa, sga)
    g1 = pltpu.async_copy(hidden_hbm.at[idx_buf.at[1]], bufb, sgb)
    g0.wait()
    w0 = pltpu.async_copy(bufa, x_hbm.at[pl.ds(offs[0], GCH)], swa)
    g1.wait()
    w1 = pltpu.async_copy(bufb, x_hbm.at[pl.ds(offs[1], GCH)], swb)
    w0.wait()
    g2 = pltpu.async_copy(hidden_hbm.at[idx_buf.at[2]], bufa, sga)
    w1.wait()
    g3 = pltpu.async_copy(hidden_hbm.at[idx_buf.at[3]], bufb, sgb)
    g2.wait()
    w2 = pltpu.async_copy(bufa, x_hbm.at[pl.ds(offs[2], GCH)], swa)
    g3.wait()
    w3 = pltpu.async_copy(bufb, x_hbm.at[pl.ds(offs[3], GCH)], swb)
    w2.wait()
    w3.wait()


_sc_gather = pl.kernel(
    _gather_body,
    out_type=[jax.ShapeDtypeStruct((NPAD, HIDDEN), jnp.float32)],
    mesh=plsc.VectorSubcoreMesh(core_axis_name="c", subcore_axis_name="s"),
    scratch_types=[
        pltpu.VMEM((4, GCH), jnp.int32),
        pltpu.VMEM((GCH, HIDDEN), jnp.float32),
        pltpu.VMEM((GCH, HIDDEN), jnp.float32),
        pltpu.SemaphoreType.DMA,
        pltpu.SemaphoreType.DMA,
        pltpu.SemaphoreType.DMA,
        pltpu.SemaphoreType.DMA,
    ],
)


def _combine_body(y_hbm, pos0_hbm, pos1_hbm, out_hbm,
                  i0, i1, p0b0, p0b1, p1b0, p1b1,
                  sa0, sb0, sa1, sb1, sw0, sw1):
    c = lax.axis_index("c")
    s = lax.axis_index("s")
    base = (c * 16 + s) * CTOK
    boffs = []
    for k in range(CTOK // CCH):
        boff = pl.multiple_of(base + k * CCH, CCH)
        boffs.append(boff)
        pltpu.sync_copy(pos0_hbm.at[pl.ds(boff, CCH)], i0.at[k])
        pltpu.sync_copy(pos1_hbm.at[pl.ds(boff, CCH)], i1.at[k])
    pairs = [(p0b0, p0b1, sa0, sb0, sw0), (p1b0, p1b1, sa1, sb1, sw1)]

    g = {}

    def start(k):
        b0, b1, sa, sb, _ = pairs[k % 2]
        g[k] = (pltpu.async_copy(y_hbm.at[i0.at[k]], b0, sa),
                pltpu.async_copy(y_hbm.at[i1.at[k]], b1, sb))

    start(0)
    start(1)
    final_w = []
    for k in range(CTOK // CCH):
        b0, b1, _, _, sw = pairs[k % 2]
        g[k][0].wait()
        g[k][1].wait()

        def arow(r, _, b0=b0, b1=b1):
            for j in range(HIDDEN // L):
                b1[r, pl.ds(j * L, L)] = (b0[r, pl.ds(j * L, L)]
                                          + b1[r, pl.ds(j * L, L)])
            return 0

        lax.fori_loop(0, CCH, arow, 0)
        wk = pltpu.async_copy(b1, out_hbm.at[pl.ds(boffs[k], CCH)], sw)
        if k + 2 < CTOK // CCH:
            wk.wait()
            start(k + 2)
        else:
            final_w.append(wk)
    for wk in final_w:
        wk.wait()


_sc_combine = pl.kernel(
    _combine_body,
    out_type=[jax.ShapeDtypeStruct((TOKENS, HIDDEN), jnp.float32)],
    mesh=plsc.VectorSubcoreMesh(core_axis_name="c", subcore_axis_name="s"),
    scratch_types=[
        pltpu.VMEM((4, CCH), jnp.int32),
        pltpu.VMEM((4, CCH), jnp.int32),
        pltpu.VMEM((CCH, HIDDEN), jnp.float32),
        pltpu.VMEM((CCH, HIDDEN), jnp.float32),
        pltpu.VMEM((CCH, HIDDEN), jnp.float32),
        pltpu.VMEM((CCH, HIDDEN), jnp.float32),
        pltpu.SemaphoreType.DMA,
        pltpu.SemaphoreType.DMA,
        pltpu.SemaphoreType.DMA,
        pltpu.SemaphoreType.DMA,
        pltpu.SemaphoreType.DMA,
        pltpu.SemaphoreType.DMA,
    ],
)


def _gemm_body(e_of_ref, nt_ref, x_ref, g_ref, u_ref, d_ref, w_ref, y_ref,
               gb_ref, ub_ref, db_ref):
    t = pl.program_id(0)
    valid = t < nt_ref[0]
    new_expert = (t == 0) | (e_of_ref[t] != e_of_ref[jnp.maximum(t - 1, 0)])

    @pl.when(new_expert & valid)
    def _cache():
        gb_ref[...] = g_ref[0].astype(jnp.bfloat16)
        ub_ref[...] = u_ref[0].astype(jnp.bfloat16)
        db_ref[...] = d_ref[0].astype(jnp.bfloat16)

    @pl.when(valid)
    def _compute():
        x = x_ref[...].astype(jnp.bfloat16)
        gate_out = jax.lax.dot_general(
            x, gb_ref[...], (((1,), (1,)), ((), ())),
            preferred_element_type=jnp.float32)
        up_out = jax.lax.dot_general(
            x, ub_ref[...], (((1,), (1,)), ((), ())),
            preferred_element_type=jnp.float32)
        act = gate_out * jax.nn.sigmoid(gate_out) * up_out
        y = jax.lax.dot_general(
            act.astype(jnp.bfloat16), db_ref[...], (((1,), (1,)), ((), ())),
            preferred_element_type=jnp.float32)
        y_ref[...] = y * w_ref[0]


def _grouped_gemm(e_of_tile, ntile, x_sorted, gate_proj, up_proj, down_proj,
                  w_sorted):
    w3 = w_sorted.reshape(GRID, TM, 1)
    grid_spec = pltpu.PrefetchScalarGridSpec(
        num_scalar_prefetch=2,
        grid=(GRID,),
        in_specs=[
            pl.BlockSpec((TM, HIDDEN), lambda t, e_of, nt: (t, 0)),
            pl.BlockSpec((1, INTER, HIDDEN),
                         lambda t, e_of, nt: (e_of[t], 0, 0)),
            pl.BlockSpec((1, INTER, HIDDEN),
                         lambda t, e_of, nt: (e_of[t], 0, 0)),
            pl.BlockSpec((1, HIDDEN, INTER),
                         lambda t, e_of, nt: (e_of[t], 0, 0)),
            pl.BlockSpec((1, TM, 1), lambda t, e_of, nt: (t, 0, 0)),
        ],
        out_specs=pl.BlockSpec((TM, HIDDEN), lambda t, e_of, nt: (t, 0)),
        scratch_shapes=[
            pltpu.VMEM((INTER, HIDDEN), jnp.bfloat16),
            pltpu.VMEM((INTER, HIDDEN), jnp.bfloat16),
            pltpu.VMEM((HIDDEN, INTER), jnp.bfloat16),
        ],
    )
    return pl.pallas_call(
        _gemm_body,
        grid_spec=grid_spec,
        out_shape=jax.ShapeDtypeStruct((NPAD, HIDDEN), jnp.float32),
        compiler_params=pltpu.CompilerParams(
            dimension_semantics=("arbitrary",)),
    )(e_of_tile, ntile, x_sorted, gate_proj, up_proj, down_proj, w3)


@jax.jit
def kernel(hidden_states, routing_weights, selected_experts, gate_proj, up_proj, down_proj):
    flat_e = selected_experts.reshape(-1).astype(jnp.int32)
    flat_w = routing_weights.reshape(-1)
    flat_tok = jnp.arange(FLAT, dtype=jnp.int32) // TOP_K

    onehot = (flat_e[:, None] == jnp.arange(NUM_EXPERTS)[None, :])
    counts = jnp.sum(onehot, axis=0, dtype=jnp.int32)
    cpad = ((counts + TM - 1) // TM) * TM
    offs_p = jnp.concatenate([jnp.zeros((1,), jnp.int32),
                              jnp.cumsum(cpad).astype(jnp.int32)])

    rank_all = jnp.cumsum(onehot.astype(jnp.int32), axis=0) - 1
    rank = jnp.sum(jnp.where(onehot, rank_all, 0), axis=1)
    dest = offs_p[flat_e] + rank

    row_sorted = jnp.zeros((NPAD,), jnp.int32).at[dest].set(flat_tok)
    w_sorted = jnp.zeros((NPAD,), jnp.float32).at[dest].set(flat_w)

    tile_starts = jnp.arange(GRID, dtype=jnp.int32) * TM
    e_of_tile = jnp.sum(tile_starts[:, None] >= offs_p[None, 1:NUM_EXPERTS],
                        axis=1).astype(jnp.int32)

    pos = dest.reshape(TOKENS, TOP_K)

    (x_sorted,) = _sc_gather(row_sorted, hidden_states)

    ntile = (offs_p[NUM_EXPERTS:NUM_EXPERTS + 1] + TM - 1) // TM

    y = _grouped_gemm(e_of_tile, ntile, x_sorted, gate_proj, up_proj,
                      down_proj, w_sorted)

    (out,) = _sc_combine(y, pos[:, 0], pos[:, 1])
    return out


# speedup vs baseline: 1.4041x; 1.1755x over previous
import functools

import jax
import jax.numpy as jnp
from jax import lax
from jax.experimental import pallas as pl
from jax.experimental.pallas import tpu as pltpu
from jax.experimental.pallas import tpu_sc as plsc

NUM_EXPERTS = 8
TOP_K = 2
HIDDEN = 1024
HHALF = HIDDEN // 2
INTER = 1408
TOKENS = 2048
FLAT = TOKENS * TOP_K

TM = 128
NPAD = FLAT + NUM_EXPERTS * TM
GRID = NPAD // TM
EOF_PAD = ((GRID + 15) // 16) * 16

L = 16
RPT = NPAD // 32
GCH = RPT // 4
CTOK = TOKENS // 32
CCH = 16


TPW = TOKENS // 32


def _dispatch_body(hidden_hbm, pos0_hbm, pos1_hbm, w0_hbm, w1_hbm,
                   x_hbm, ws_hbm, hbuf, d0, d1, wb0, wb1, s0, s1, s2, s3):
    c = lax.axis_index("c")
    s = lax.axis_index("s")
    base = pl.multiple_of((c * 16 + s) * TPW, TPW)
    pltpu.sync_copy(pos0_hbm.at[pl.ds(base, TPW)], d0)
    pltpu.sync_copy(pos1_hbm.at[pl.ds(base, TPW)], d1)
    pltpu.sync_copy(w0_hbm.at[pl.ds(base, TPW)], wb0)
    pltpu.sync_copy(w1_hbm.at[pl.ds(base, TPW)], wb1)
    cw0 = pltpu.async_copy(wb0, ws_hbm.at[d0], s2)
    cw1 = pltpu.async_copy(wb1, ws_hbm.at[d1], s3)
    pltpu.sync_copy(hidden_hbm.at[pl.ds(base, TPW)], hbuf)
    cx0 = pltpu.async_copy(hbuf, x_hbm.at[d0], s0)
    cx1 = pltpu.async_copy(hbuf, x_hbm.at[d1], s1)
    cw0.wait()
    cw1.wait()
    cx0.wait()
    cx1.wait()


_sc_dispatch = pl.kernel(
    _dispatch_body,
    out_type=[
        jax.ShapeDtypeStruct((NPAD, HIDDEN), jnp.float32),
        jax.ShapeDtypeStruct((NPAD,), jnp.float32),
    ],
    mesh=plsc.VectorSubcoreMesh(core_axis_name="c", subcore_axis_name="s"),
    scratch_types=[
        pltpu.VMEM((TPW, HIDDEN), jnp.float32),
        pltpu.VMEM((TPW,), jnp.int32),
        pltpu.VMEM((TPW,), jnp.int32),
        pltpu.VMEM((TPW,), jnp.float32),
        pltpu.VMEM((TPW,), jnp.float32),
        pltpu.SemaphoreType.DMA,
        pltpu.SemaphoreType.DMA,
        pltpu.SemaphoreType.DMA,
        pltpu.SemaphoreType.DMA,
    ],
)


def _combine_body(y_hbm, pos0_hbm, pos1_hbm, out_hbm,
                  i0, i1, p0b0, p0b1, p1b0, p1b1,
                  sa0, sb0, sa1, sb1, sw0, sw1):
    c = lax.axis_index("c")
    s = lax.axis_index("s")
    base = (c * 16 + s) * CTOK
    boffs = []
    for k in range(CTOK // CCH):
        boff = pl.multiple_of(base + k * CCH, CCH)
        boffs.append(boff)
        pltpu.sync_copy(pos0_hbm.at[pl.ds(boff, CCH)], i0.at[k])
        pltpu.sync_copy(pos1_hbm.at[pl.ds(boff, CCH)], i1.at[k])
    pairs = [(p0b0, p0b1, sa0, sb0, sw0), (p1b0, p1b1, sa1, sb1, sw1)]

    g = {}

    def start(k):
        b0, b1, sa, sb, _ = pairs[k % 2]
        g[k] = (pltpu.async_copy(y_hbm.at[i0.at[k]], b0, sa),
                pltpu.async_copy(y_hbm.at[i1.at[k]], b1, sb))

    start(0)
    start(1)
    final_w = []
    for k in range(CTOK // CCH):
        b0, b1, _, _, sw = pairs[k % 2]
        g[k][0].wait()
        g[k][1].wait()

        def arow(r, _, b0=b0, b1=b1):
            for j in range(HIDDEN // L):
                b1[r, pl.ds(j * L, L)] = (b0[r, pl.ds(j * L, L)]
                                          + b1[r, pl.ds(j * L, L)])
            return 0

        lax.fori_loop(0, CCH, arow, 0)
        wk = pltpu.async_copy(b1, out_hbm.at[pl.ds(boffs[k], CCH)], sw)
        if k + 2 < CTOK // CCH:
            wk.wait()
            start(k + 2)
        else:
            final_w.append(wk)
    for wk in final_w:
        wk.wait()


_sc_combine = pl.kernel(
    _combine_body,
    out_type=[jax.ShapeDtypeStruct((TOKENS, HIDDEN), jnp.float32)],
    mesh=plsc.VectorSubcoreMesh(core_axis_name="c", subcore_axis_name="s"),
    scratch_types=[
        pltpu.VMEM((4, CCH), jnp.int32),
        pltpu.VMEM((4, CCH), jnp.int32),
        pltpu.VMEM((CCH, HIDDEN), jnp.float32),
        pltpu.VMEM((CCH, HIDDEN), jnp.float32),
        pltpu.VMEM((CCH, HIDDEN), jnp.float32),
        pltpu.VMEM((CCH, HIDDEN), jnp.float32),
        pltpu.SemaphoreType.DMA,
        pltpu.SemaphoreType.DMA,
        pltpu.SemaphoreType.DMA,
        pltpu.SemaphoreType.DMA,
        pltpu.SemaphoreType.DMA,
        pltpu.SemaphoreType.DMA,
    ],
)


def _gemm_body(e_of_ref, nt_ref, vend_ref, x_ref, g_ref, u_ref, d_ref,
               w_ref, y_ref, gb_ref, ub_ref, db_ref):
    t = pl.program_id(0)
    valid = t < nt_ref[0]
    new_expert = (t == 0) | (e_of_ref[t] != e_of_ref[jnp.maximum(t - 1, 0)])

    @pl.when(new_expert & valid)
    def _cache():
        gb_ref[...] = g_ref[0].astype(jnp.bfloat16)
        ub_ref[...] = u_ref[0].astype(jnp.bfloat16)
        db_ref[...] = d_ref[0].astype(jnp.bfloat16)

    @pl.when(valid)
    def _compute():
        x = x_ref[...].astype(jnp.bfloat16)
        gate_out = jax.lax.dot_general(
            x, gb_ref[...], (((1,), (1,)), ((), ())),
            preferred_element_type=jnp.float32)
        up_out = jax.lax.dot_general(
            x, ub_ref[...], (((1,), (1,)), ((), ())),
            preferred_element_type=jnp.float32)
        act = gate_out * jax.nn.sigmoid(gate_out) * up_out
        y = jax.lax.dot_general(
            act.astype(jnp.bfloat16), db_ref[...], (((1,), (1,)), ((), ())),
            preferred_element_type=jnp.float32)
        rows = jax.lax.broadcasted_iota(jnp.int32, (TM, 1), 0)
        rmask = (rows < (vend_ref[t] - t * TM)).astype(jnp.float32)
        y_ref[...] = y * (w_ref[0] * rmask)


def _grouped_gemm(e_of_tile, ntile, vend, x_sorted, gate_proj, up_proj,
                  down_proj, w_sorted):
    w3 = w_sorted.reshape(GRID, TM, 1)
    grid_spec = pltpu.PrefetchScalarGridSpec(
        num_scalar_prefetch=3,
        grid=(GRID,),
        in_specs=[
            pl.BlockSpec((TM, HIDDEN), lambda t, e_of, nt, vend: (t, 0)),
            pl.BlockSpec((1, INTER, HIDDEN),
                         lambda t, e_of, nt, vend: (e_of[t], 0, 0)),
            pl.BlockSpec((1, INTER, HIDDEN),
                         lambda t, e_of, nt, vend: (e_of[t], 0, 0)),
            pl.BlockSpec((1, HIDDEN, INTER),
                         lambda t, e_of, nt, vend: (e_of[t], 0, 0)),
            pl.BlockSpec((1, TM, 1), lambda t, e_of, nt, vend: (t, 0, 0)),
        ],
        out_specs=pl.BlockSpec((TM, HIDDEN), lambda t, e_of, nt, vend: (t, 0)),
        scratch_shapes=[
            pltpu.VMEM((INTER, HIDDEN), jnp.bfloat16),
            pltpu.VMEM((INTER, HIDDEN), jnp.bfloat16),
            pltpu.VMEM((HIDDEN, INTER), jnp.bfloat16),
        ],
    )
    return pl.pallas_call(
        _gemm_body,
        grid_spec=grid_spec,
        out_shape=jax.ShapeDtypeStruct((NPAD, HIDDEN), jnp.float32),
        compiler_params=pltpu.CompilerParams(
            dimension_semantics=("arbitrary",)),
    )(e_of_tile, ntile, vend, x_sorted, gate_proj, up_proj, down_proj, w3)


@jax.jit
def kernel(hidden_states, routing_weights, selected_experts, gate_proj, up_proj, down_proj):
    flat_e = selected_experts.reshape(-1).astype(jnp.int32)

    onehot = (flat_e[:, None] == jnp.arange(NUM_EXPERTS)[None, :])
    counts = jnp.sum(onehot, axis=0, dtype=jnp.int32)
    cpad = ((counts + TM - 1) // TM) * TM
    offs_p = jnp.concatenate([jnp.zeros((1,), jnp.int32),
                              jnp.cumsum(cpad).astype(jnp.int32)])

    rank_all = jnp.cumsum(onehot.astype(jnp.int32), axis=0) - 1
    rank = jnp.sum(jnp.where(onehot, rank_all, 0), axis=1)
    dest = offs_p[flat_e] + rank
    pos = dest.reshape(TOKENS, TOP_K)

    tile_starts = jnp.arange(GRID, dtype=jnp.int32) * TM
    e_of_tile = jnp.sum(tile_starts[:, None] >= offs_p[None, 1:NUM_EXPERTS],
                        axis=1).astype(jnp.int32)
    ntile = (offs_p[NUM_EXPERTS:NUM_EXPERTS + 1] + TM - 1) // TM
    vend = offs_p[e_of_tile] + counts[e_of_tile]

    x_sorted, w_sorted = _sc_dispatch(
        hidden_states, pos[:, 0], pos[:, 1],
        routing_weights[:, 0], routing_weights[:, 1])

    y = _grouped_gemm(e_of_tile, ntile, vend, x_sorted, gate_proj, up_proj,
                      down_proj, w_sorted)

    (out,) = _sc_combine(y, pos[:, 0], pos[:, 1])
    return out


# speedup vs baseline: 1.7029x; 1.2128x over previous
import functools

import jax
import jax.numpy as jnp
from jax import lax
from jax.experimental import pallas as pl
from jax.experimental.pallas import tpu as pltpu
from jax.experimental.pallas import tpu_sc as plsc

NUM_EXPERTS = 8
TOP_K = 2
HIDDEN = 1024
HHALF = HIDDEN // 2
INTER = 1408
TOKENS = 2048
FLAT = TOKENS * TOP_K

TM = 256
NPAD = FLAT + NUM_EXPERTS * TM
GRID = NPAD // TM
EOF_PAD = ((GRID + 15) // 16) * 16

L = 16
RPT = NPAD // 32
GCH = RPT // 4
CTOK = TOKENS // 32
CCH = 16


TPW = TOKENS // 32


def _dispatch_body(hidden_hbm, pos0_hbm, pos1_hbm, w0_hbm, w1_hbm,
                   x_hbm, ws_hbm, hbuf, d0, d1, wb0, wb1, s0, s1, s2, s3):
    c = lax.axis_index("c")
    s = lax.axis_index("s")
    base = pl.multiple_of((c * 16 + s) * TPW, TPW)
    pltpu.sync_copy(pos0_hbm.at[pl.ds(base, TPW)], d0)
    pltpu.sync_copy(pos1_hbm.at[pl.ds(base, TPW)], d1)
    pltpu.sync_copy(w0_hbm.at[pl.ds(base, TPW)], wb0)
    pltpu.sync_copy(w1_hbm.at[pl.ds(base, TPW)], wb1)
    cw0 = pltpu.async_copy(wb0, ws_hbm.at[d0], s2)
    cw1 = pltpu.async_copy(wb1, ws_hbm.at[d1], s3)
    pltpu.sync_copy(hidden_hbm.at[pl.ds(base, TPW)], hbuf)
    cx0 = pltpu.async_copy(hbuf, x_hbm.at[d0], s0)
    cx1 = pltpu.async_copy(hbuf, x_hbm.at[d1], s1)
    cw0.wait()
    cw1.wait()
    cx0.wait()
    cx1.wait()


_sc_dispatch = pl.kernel(
    _dispatch_body,
    out_type=[
        jax.ShapeDtypeStruct((NPAD, HIDDEN), jnp.float32),
        jax.ShapeDtypeStruct((NPAD,), jnp.float32),
    ],
    mesh=plsc.VectorSubcoreMesh(core_axis_name="c", subcore_axis_name="s"),
    scratch_types=[
        pltpu.VMEM((TPW, HIDDEN), jnp.float32),
        pltpu.VMEM((TPW,), jnp.int32),
        pltpu.VMEM((TPW,), jnp.int32),
        pltpu.VMEM((TPW,), jnp.float32),
        pltpu.VMEM((TPW,), jnp.float32),
        pltpu.SemaphoreType.DMA,
        pltpu.SemaphoreType.DMA,
        pltpu.SemaphoreType.DMA,
        pltpu.SemaphoreType.DMA,
    ],
)


def _combine_body(y_hbm, pos0_hbm, pos1_hbm, out_hbm,
                  i0, i1, p0b0, p0b1, p1b0, p1b1,
                  sa0, sb0, sa1, sb1, sw0, sw1):
    c = lax.axis_index("c")
    s = lax.axis_index("s")
    base = (c * 16 + s) * CTOK
    boffs = []
    for k in range(CTOK // CCH):
        boff = pl.multiple_of(base + k * CCH, CCH)
        boffs.append(boff)
        pltpu.sync_copy(pos0_hbm.at[pl.ds(boff, CCH)], i0.at[k])
        pltpu.sync_copy(pos1_hbm.at[pl.ds(boff, CCH)], i1.at[k])
    pairs = [(p0b0, p0b1, sa0, sb0, sw0), (p1b0, p1b1, sa1, sb1, sw1)]

    g = {}

    def start(k):
        b0, b1, sa, sb, _ = pairs[k % 2]
        g[k] = (pltpu.async_copy(y_hbm.at[i0.at[k]], b0, sa),
                pltpu.async_copy(y_hbm.at[i1.at[k]], b1, sb))

    start(0)
    start(1)
    final_w = []
    for k in range(CTOK // CCH):
        b0, b1, _, _, sw = pairs[k % 2]
        g[k][0].wait()
        g[k][1].wait()

        def arow(r, _, b0=b0, b1=b1):
            for j in range(HIDDEN // L):
                b1[r, pl.ds(j * L, L)] = (b0[r, pl.ds(j * L, L)]
                                          + b1[r, pl.ds(j * L, L)])
            return 0

        lax.fori_loop(0, CCH, arow, 0)
        wk = pltpu.async_copy(b1, out_hbm.at[pl.ds(boffs[k], CCH)], sw)
        if k + 2 < CTOK // CCH:
            wk.wait()
            start(k + 2)
        else:
            final_w.append(wk)
    for wk in final_w:
        wk.wait()


_sc_combine = pl.kernel(
    _combine_body,
    out_type=[jax.ShapeDtypeStruct((TOKENS, HIDDEN), jnp.float32)],
    mesh=plsc.VectorSubcoreMesh(core_axis_name="c", subcore_axis_name="s"),
    scratch_types=[
        pltpu.VMEM((4, CCH), jnp.int32),
        pltpu.VMEM((4, CCH), jnp.int32),
        pltpu.VMEM((CCH, HIDDEN), jnp.float32),
        pltpu.VMEM((CCH, HIDDEN), jnp.float32),
        pltpu.VMEM((CCH, HIDDEN), jnp.float32),
        pltpu.VMEM((CCH, HIDDEN), jnp.float32),
        pltpu.SemaphoreType.DMA,
        pltpu.SemaphoreType.DMA,
        pltpu.SemaphoreType.DMA,
        pltpu.SemaphoreType.DMA,
        pltpu.SemaphoreType.DMA,
        pltpu.SemaphoreType.DMA,
    ],
)


def _gemm_body(e_of_ref, nt_ref, vend_ref, x_ref, g_ref, u_ref, d_ref,
               w_ref, y_ref, gb_ref, ub_ref, db_ref):
    t = pl.program_id(0)
    valid = t < nt_ref[0]
    new_expert = (t == 0) | (e_of_ref[t] != e_of_ref[jnp.maximum(t - 1, 0)])

    @pl.when(new_expert & valid)
    def _cache():
        gb_ref[...] = g_ref[0].astype(jnp.bfloat16)
        ub_ref[...] = u_ref[0].astype(jnp.bfloat16)
        db_ref[...] = d_ref[0].astype(jnp.bfloat16)

    @pl.when(valid)
    def _compute():
        x = x_ref[...].astype(jnp.bfloat16)
        gate_out = jax.lax.dot_general(
            x, gb_ref[...], (((1,), (1,)), ((), ())),
            preferred_element_type=jnp.float32)
        up_out = jax.lax.dot_general(
            x, ub_ref[...], (((1,), (1,)), ((), ())),
            preferred_element_type=jnp.float32)
        act = gate_out * jax.nn.sigmoid(gate_out) * up_out
        y = jax.lax.dot_general(
            act.astype(jnp.bfloat16), db_ref[...], (((1,), (1,)), ((), ())),
            preferred_element_type=jnp.float32)
        rows = jax.lax.broadcasted_iota(jnp.int32, (TM, 1), 0)
        rmask = (rows < (vend_ref[t] - t * TM)).astype(jnp.float32)
        y_ref[...] = y * (w_ref[0] * rmask)


def _grouped_gemm(e_of_tile, ntile, vend, x_sorted, gate_proj, up_proj,
                  down_proj, w_sorted):
    w3 = w_sorted.reshape(GRID, TM, 1)
    grid_spec = pltpu.PrefetchScalarGridSpec(
        num_scalar_prefetch=3,
        grid=(GRID,),
        in_specs=[
            pl.BlockSpec((TM, HIDDEN), lambda t, e_of, nt, vend: (t, 0)),
            pl.BlockSpec((1, INTER, HIDDEN),
                         lambda t, e_of, nt, vend: (e_of[t], 0, 0)),
            pl.BlockSpec((1, INTER, HIDDEN),
                         lambda t, e_of, nt, vend: (e_of[t], 0, 0)),
            pl.BlockSpec((1, HIDDEN, INTER),
                         lambda t, e_of, nt, vend: (e_of[t], 0, 0)),
            pl.BlockSpec((1, TM, 1), lambda t, e_of, nt, vend: (t, 0, 0)),
        ],
        out_specs=pl.BlockSpec((TM, HIDDEN), lambda t, e_of, nt, vend: (t, 0)),
        scratch_shapes=[
            pltpu.VMEM((INTER, HIDDEN), jnp.bfloat16),
            pltpu.VMEM((INTER, HIDDEN), jnp.bfloat16),
            pltpu.VMEM((HIDDEN, INTER), jnp.bfloat16),
        ],
    )
    return pl.pallas_call(
        _gemm_body,
        grid_spec=grid_spec,
        out_shape=jax.ShapeDtypeStruct((NPAD, HIDDEN), jnp.float32),
        compiler_params=pltpu.CompilerParams(
            dimension_semantics=("arbitrary",)),
    )(e_of_tile, ntile, vend, x_sorted, gate_proj, up_proj, down_proj, w3)


@jax.jit
def kernel(hidden_states, routing_weights, selected_experts, gate_proj, up_proj, down_proj):
    flat_e = selected_experts.reshape(-1).astype(jnp.int32)

    onehot = (flat_e[:, None] == jnp.arange(NUM_EXPERTS)[None, :])
    counts = jnp.sum(onehot, axis=0, dtype=jnp.int32)
    cpad = ((counts + TM - 1) // TM) * TM
    offs_p = jnp.concatenate([jnp.zeros((1,), jnp.int32),
                              jnp.cumsum(cpad).astype(jnp.int32)])

    rank_all = jnp.cumsum(onehot.astype(jnp.int32), axis=0) - 1
    rank = jnp.sum(jnp.where(onehot, rank_all, 0), axis=1)
    dest = offs_p[flat_e] + rank
    pos = dest.reshape(TOKENS, TOP_K)

    tile_starts = jnp.arange(GRID, dtype=jnp.int32) * TM
    e_of_tile = jnp.sum(tile_starts[:, None] >= offs_p[None, 1:NUM_EXPERTS],
                        axis=1).astype(jnp.int32)
    ntile = (offs_p[NUM_EXPERTS:NUM_EXPERTS + 1] + TM - 1) // TM
    vend = offs_p[e_of_tile] + counts[e_of_tile]

    x_sorted, w_sorted = _sc_dispatch(
        hidden_states, pos[:, 0], pos[:, 1],
        routing_weights[:, 0], routing_weights[:, 1])

    y = _grouped_gemm(e_of_tile, ntile, vend, x_sorted, gate_proj, up_proj,
                      down_proj, w_sorted)

    (out,) = _sc_combine(y, pos[:, 0], pos[:, 1])
    return out


# speedup vs baseline: 1.8428x; 1.0821x over previous
import functools

import jax
import jax.numpy as jnp
from jax import lax
from jax.experimental import pallas as pl
from jax.experimental.pallas import tpu as pltpu
from jax.experimental.pallas import tpu_sc as plsc

NUM_EXPERTS = 8
TOP_K = 2
HIDDEN = 1024
HHALF = HIDDEN // 2
INTER = 1408
TOKENS = 2048
FLAT = TOKENS * TOP_K

TM = 512
NPAD = FLAT + NUM_EXPERTS * TM
GRID = NPAD // TM
EOF_PAD = ((GRID + 15) // 16) * 16

L = 16
RPT = NPAD // 32
GCH = RPT // 4
CTOK = TOKENS // 32
CCH = 16


TPW = TOKENS // 32


def _dispatch_body(hidden_hbm, pos0_hbm, pos1_hbm, w0_hbm, w1_hbm,
                   x_hbm, ws_hbm, hbuf, d0, d1, wb0, wb1, s0, s1, s2, s3):
    c = lax.axis_index("c")
    s = lax.axis_index("s")
    base = pl.multiple_of((c * 16 + s) * TPW, TPW)
    pltpu.sync_copy(pos0_hbm.at[pl.ds(base, TPW)], d0)
    pltpu.sync_copy(pos1_hbm.at[pl.ds(base, TPW)], d1)
    pltpu.sync_copy(w0_hbm.at[pl.ds(base, TPW)], wb0)
    pltpu.sync_copy(w1_hbm.at[pl.ds(base, TPW)], wb1)
    cw0 = pltpu.async_copy(wb0, ws_hbm.at[d0], s2)
    cw1 = pltpu.async_copy(wb1, ws_hbm.at[d1], s3)
    pltpu.sync_copy(hidden_hbm.at[pl.ds(base, TPW)], hbuf)
    cx0 = pltpu.async_copy(hbuf, x_hbm.at[d0], s0)
    cx1 = pltpu.async_copy(hbuf, x_hbm.at[d1], s1)
    cw0.wait()
    cw1.wait()
    cx0.wait()
    cx1.wait()


_sc_dispatch = pl.kernel(
    _dispatch_body,
    out_type=[
        jax.ShapeDtypeStruct((NPAD, HIDDEN), jnp.float32),
        jax.ShapeDtypeStruct((NPAD,), jnp.float32),
    ],
    mesh=plsc.VectorSubcoreMesh(core_axis_name="c", subcore_axis_name="s"),
    scratch_types=[
        pltpu.VMEM((TPW, HIDDEN), jnp.float32),
        pltpu.VMEM((TPW,), jnp.int32),
        pltpu.VMEM((TPW,), jnp.int32),
        pltpu.VMEM((TPW,), jnp.float32),
        pltpu.VMEM((TPW,), jnp.float32),
        pltpu.SemaphoreType.DMA,
        pltpu.SemaphoreType.DMA,
        pltpu.SemaphoreType.DMA,
        pltpu.SemaphoreType.DMA,
    ],
)


def _combine_body(y_hbm, pos0_hbm, pos1_hbm, out_hbm,
                  i0, i1, p0b0, p0b1, p1b0, p1b1,
                  sa0, sb0, sa1, sb1, sw0, sw1):
    c = lax.axis_index("c")
    s = lax.axis_index("s")
    base = (c * 16 + s) * CTOK
    boffs = []
    for k in range(CTOK // CCH):
        boff = pl.multiple_of(base + k * CCH, CCH)
        boffs.append(boff)
        pltpu.sync_copy(pos0_hbm.at[pl.ds(boff, CCH)], i0.at[k])
        pltpu.sync_copy(pos1_hbm.at[pl.ds(boff, CCH)], i1.at[k])
    pairs = [(p0b0, p0b1, sa0, sb0, sw0), (p1b0, p1b1, sa1, sb1, sw1)]

    g = {}

    def start(k):
        b0, b1, sa, sb, _ = pairs[k % 2]
        g[k] = (pltpu.async_copy(y_hbm.at[i0.at[k]], b0, sa),
                pltpu.async_copy(y_hbm.at[i1.at[k]], b1, sb))

    start(0)
    start(1)
    final_w = []
    for k in range(CTOK // CCH):
        b0, b1, _, _, sw = pairs[k % 2]
        g[k][0].wait()
        g[k][1].wait()

        def arow(r, _, b0=b0, b1=b1):
            for j in range(HIDDEN // L):
                b1[r, pl.ds(j * L, L)] = (b0[r, pl.ds(j * L, L)]
                                          + b1[r, pl.ds(j * L, L)])
            return 0

        lax.fori_loop(0, CCH, arow, 0)
        wk = pltpu.async_copy(b1, out_hbm.at[pl.ds(boffs[k], CCH)], sw)
        if k + 2 < CTOK // CCH:
            wk.wait()
            start(k + 2)
        else:
            final_w.append(wk)
    for wk in final_w:
        wk.wait()


_sc_combine = pl.kernel(
    _combine_body,
    out_type=[jax.ShapeDtypeStruct((TOKENS, HIDDEN), jnp.float32)],
    mesh=plsc.VectorSubcoreMesh(core_axis_name="c", subcore_axis_name="s"),
    scratch_types=[
        pltpu.VMEM((4, CCH), jnp.int32),
        pltpu.VMEM((4, CCH), jnp.int32),
        pltpu.VMEM((CCH, HIDDEN), jnp.float32),
        pltpu.VMEM((CCH, HIDDEN), jnp.float32),
        pltpu.VMEM((CCH, HIDDEN), jnp.float32),
        pltpu.VMEM((CCH, HIDDEN), jnp.float32),
        pltpu.SemaphoreType.DMA,
        pltpu.SemaphoreType.DMA,
        pltpu.SemaphoreType.DMA,
        pltpu.SemaphoreType.DMA,
        pltpu.SemaphoreType.DMA,
        pltpu.SemaphoreType.DMA,
    ],
)


def _gemm_body(e_of_ref, nt_ref, vend_ref, x_ref, g_ref, u_ref, d_ref,
               w_ref, y_ref, gb_ref, ub_ref, db_ref):
    t = pl.program_id(0)
    valid = t < nt_ref[0]
    new_expert = (t == 0) | (e_of_ref[t] != e_of_ref[jnp.maximum(t - 1, 0)])

    @pl.when(new_expert & valid)
    def _cache():
        gb_ref[...] = g_ref[0].astype(jnp.bfloat16)
        ub_ref[...] = u_ref[0].astype(jnp.bfloat16)
        db_ref[...] = d_ref[0].astype(jnp.bfloat16)

    @pl.when(valid)
    def _compute():
        x = x_ref[...].astype(jnp.bfloat16)
        gate_out = jax.lax.dot_general(
            x, gb_ref[...], (((1,), (1,)), ((), ())),
            preferred_element_type=jnp.float32)
        up_out = jax.lax.dot_general(
            x, ub_ref[...], (((1,), (1,)), ((), ())),
            preferred_element_type=jnp.float32)
        act = gate_out * jax.nn.sigmoid(gate_out) * up_out
        y = jax.lax.dot_general(
            act.astype(jnp.bfloat16), db_ref[...], (((1,), (1,)), ((), ())),
            preferred_element_type=jnp.float32)
        rows = jax.lax.broadcasted_iota(jnp.int32, (TM, 1), 0)
        rmask = (rows < (vend_ref[t] - t * TM)).astype(jnp.float32)
        y_ref[...] = y * (w_ref[0] * rmask)


def _grouped_gemm(e_of_tile, ntile, vend, x_sorted, gate_proj, up_proj,
                  down_proj, w_sorted):
    w3 = w_sorted.reshape(GRID, TM, 1)
    grid_spec = pltpu.PrefetchScalarGridSpec(
        num_scalar_prefetch=3,
        grid=(GRID,),
        in_specs=[
            pl.BlockSpec((TM, HIDDEN), lambda t, e_of, nt, vend: (t, 0)),
            pl.BlockSpec((1, INTER, HIDDEN),
                         lambda t, e_of, nt, vend: (e_of[t], 0, 0)),
            pl.BlockSpec((1, INTER, HIDDEN),
                         lambda t, e_of, nt, vend: (e_of[t], 0, 0)),
            pl.BlockSpec((1, HIDDEN, INTER),
                         lambda t, e_of, nt, vend: (e_of[t], 0, 0)),
            pl.BlockSpec((1, TM, 1), lambda t, e_of, nt, vend: (t, 0, 0)),
        ],
        out_specs=pl.BlockSpec((TM, HIDDEN), lambda t, e_of, nt, vend: (t, 0)),
        scratch_shapes=[
            pltpu.VMEM((INTER, HIDDEN), jnp.bfloat16),
            pltpu.VMEM((INTER, HIDDEN), jnp.bfloat16),
            pltpu.VMEM((HIDDEN, INTER), jnp.bfloat16),
        ],
    )
    return pl.pallas_call(
        _gemm_body,
        grid_spec=grid_spec,
        out_shape=jax.ShapeDtypeStruct((NPAD, HIDDEN), jnp.float32),
        compiler_params=pltpu.CompilerParams(
            dimension_semantics=("arbitrary",)),
    )(e_of_tile, ntile, vend, x_sorted, gate_proj, up_proj, down_proj, w3)


@jax.jit
def kernel(hidden_states, routing_weights, selected_experts, gate_proj, up_proj, down_proj):
    flat_e = selected_experts.reshape(-1).astype(jnp.int32)

    onehot = (flat_e[:, None] == jnp.arange(NUM_EXPERTS)[None, :])
    counts = jnp.sum(onehot, axis=0, dtype=jnp.int32)
    cpad = ((counts + TM - 1) // TM) * TM
    offs_p = jnp.concatenate([jnp.zeros((1,), jnp.int32),
                              jnp.cumsum(cpad).astype(jnp.int32)])

    rank_all = jnp.cumsum(onehot.astype(jnp.int32), axis=0) - 1
    rank = jnp.sum(jnp.where(onehot, rank_all, 0), axis=1)
    dest = offs_p[flat_e] + rank
    pos = dest.reshape(TOKENS, TOP_K)

    tile_starts = jnp.arange(GRID, dtype=jnp.int32) * TM
    e_of_tile = jnp.sum(tile_starts[:, None] >= offs_p[None, 1:NUM_EXPERTS],
                        axis=1).astype(jnp.int32)
    ntile = (offs_p[NUM_EXPERTS:NUM_EXPERTS + 1] + TM - 1) // TM
    vend = offs_p[e_of_tile] + counts[e_of_tile]

    x_sorted, w_sorted = _sc_dispatch(
        hidden_states, pos[:, 0], pos[:, 1],
        routing_weights[:, 0], routing_weights[:, 1])

    y = _grouped_gemm(e_of_tile, ntile, vend, x_sorted, gate_proj, up_proj,
                      down_proj, w_sorted)

    (out,) = _sc_combine(y, pos[:, 0], pos[:, 1])
    return out
